# baseline (speedup 1.0000x reference)
import functools

import jax
import jax.numpy as jnp
from jax import lax
from jax.experimental import pallas as pl
from jax.experimental.pallas import tpu as pltpu
from jax.experimental.pallas import tpu_sc as plsc

L = 16
NC = 2
NS = 16
NW = NC * NS
BIG = 0x3FFFFFFF
CHUNK = 2000


def _tile_body(preds_hbm, lits_hbm, clauses_hbm, out_hbm,
               preds_v, lits_b0, lits_b1, cls_b0, cls_b1, sbuf, outbuf_v,
               sem_p, sem_l0, sem_l1, sem_c0, sem_c1,
               *, n_vars, per_tile):
    wid = lax.axis_index("s") * NC + lax.axis_index("c")
    base = wid * per_tile
    nchunk = per_tile // CHUNK
    lits_b = (lits_b0, lits_b1)
    cls_b = (cls_b0, cls_b1)
    sem_l = (sem_l0, sem_l1)
    sem_c = (sem_c0, sem_c1)

    def start_chunk(slot, c):
        off = base + c * CHUNK
        pltpu.make_async_copy(lits_hbm.at[pl.ds(off, CHUNK)],
                              lits_b[slot], sem_l[slot]).start()
        pltpu.make_async_copy(clauses_hbm.at[pl.ds(off, CHUNK)],
                              cls_b[slot], sem_c[slot]).start()

    def wait_chunk(slot):
        pltpu.make_async_copy(lits_hbm.at[pl.ds(0, CHUNK)],
                              lits_b[slot], sem_l[slot]).wait()
        pltpu.make_async_copy(clauses_hbm.at[pl.ds(0, CHUNK)],
                              cls_b[slot], sem_c[slot]).wait()

    preds_cp = pltpu.make_async_copy(preds_hbm, preds_v, sem_p)
    preds_cp.start()
    start_chunk(0, 0)
    start_chunk(1, 1)
    preds_cp.wait()

    iota = lax.iota(jnp.int32, L)
    lane0 = iota == 0
    zero16 = jnp.zeros((L,), jnp.int32)
    one16 = jnp.full((L,), 1, jnp.int32)

    plsc.store_scatter(sbuf, [zero16], jnp.full((L,), -1, jnp.int32),
                       mask=lane0)

    def compute(slot, state):
        cnt_vec, first_vec, last_vec = state

        def vec_body(i, ptr):
            lit = lits_b[slot][pl.ds(i * L, L)]
            cls = cls_b[slot][pl.ds(i * L, L)]
            is_pos = lit < n_vars
            var = jnp.where(is_pos, lit, lit - n_vars)
            p = plsc.load_gather(preds_v, [var])
            sat = (p >= 0.5) == is_pos
            plsc.store_compressed(sbuf.at[pl.ds(ptr, L)], cls, mask=sat)
            pc = plsc.all_reduce_population_count(sat)
            return ptr + pc[0]

        ptr = lax.fori_loop(0, CHUNK // L, vec_body, jnp.int32(1), unroll=4)
        n = ptr - 1

        def p2_body(j, cv):
            k = jnp.minimum(j * L + iota, n - 1)
            x = plsc.load_gather(sbuf, [k + 1])
            xm = plsc.load_gather(sbuf, [k])
            valid = (j * L + iota) < n
            return cv + (valid & (x != xm)).astype(jnp.int32)

        cnt_vec = lax.fori_loop(0, (n + L - 1) // L, p2_body, cnt_vec)

        firstv = plsc.load_gather(sbuf, [one16])
        first_vec = jnp.minimum(first_vec, jnp.where(n > 0, firstv, BIG))
        lastv = plsc.load_gather(sbuf, [zero16 + n])
        plsc.store_scatter(sbuf, [zero16], lastv, mask=lane0)
        last_vec = jnp.where(n > 0, lastv, last_vec)
        return cnt_vec, first_vec, last_vec

    def one(c, slot, state):
        wait_chunk(slot)
        state = compute(slot, state)

        @pl.when(c + 2 < nchunk)
        def _():
            start_chunk(slot, c + 2)

        return state

    def pair_body(i, state):
        state = one(2 * i, 0, state)
        state = one(2 * i + 1, 1, state)
        return state

    init = (jnp.zeros((L,), jnp.int32), jnp.full((L,), BIG, jnp.int32),
            jnp.full((L,), -1, jnp.int32))
    cnt_vec, first_vec, last_vec = lax.fori_loop(
        0, nchunk // 2, pair_body, init)

    cnt = jnp.sum(cnt_vec)
    first = jnp.min(first_vec)
    out = jnp.where(iota == 0, cnt,
                    jnp.where(iota == 1, first,
                              jnp.where(iota == 2, last_vec, 0)))
    outbuf_v[...] = out
    pltpu.sync_copy(outbuf_v, out_hbm.at[wid])


def _combine_body(n_vars, partials_ref, ncl_ref, o_ref):
    def body(t, st):
        total, m = st
        c = partials_ref[t, 0]
        f = partials_ref[t, 1]
        l = partials_ref[t, 2]
        dup = jnp.where((c > 0) & (f == m), jnp.int32(1), jnp.int32(0))
        return total + c - dup, jnp.maximum(m, l)

    total, _ = lax.fori_loop(0, NW, body, (jnp.int32(0), jnp.int32(-1)))
    o_ref[0, 0] = (ncl_ref[0, 0] - total.astype(jnp.float32)) / jnp.float32(n_vars)


def kernel(preds, lits, clauses, n_vars, n_clauses):
    del n_vars
    nv = preds.shape[0]
    nnz = lits.shape[0]
    per_tile = nnz // NW
    assert nnz % NW == 0
    assert per_tile % (2 * CHUNK) == 0 and CHUNK % L == 0

    mesh = plsc.VectorSubcoreMesh(core_axis_name="c", subcore_axis_name="s")
    sc = functools.partial(
        pl.kernel,
        mesh=mesh,
        compiler_params=pltpu.CompilerParams(needs_layout_passes=False),
        out_type=jax.ShapeDtypeStruct((NW, L), jnp.int32),
        scratch_types=[
            pltpu.VMEM((nv,), jnp.float32),
            pltpu.VMEM((CHUNK,), jnp.int32),
            pltpu.VMEM((CHUNK,), jnp.int32),
            pltpu.VMEM((CHUNK,), jnp.int32),
            pltpu.VMEM((CHUNK,), jnp.int32),
            pltpu.VMEM((CHUNK + L + 8,), jnp.int32),
            pltpu.VMEM((L,), jnp.int32),
            pltpu.SemaphoreType.DMA,
            pltpu.SemaphoreType.DMA,
            pltpu.SemaphoreType.DMA,
            pltpu.SemaphoreType.DMA,
            pltpu.SemaphoreType.DMA,
        ],
    )(functools.partial(_tile_body, n_vars=nv, per_tile=per_tile))
    partials = sc(preds, lits, clauses)

    ncl = jnp.asarray(n_clauses, jnp.float32).reshape(1, 1)
    out = pl.pallas_call(
        functools.partial(_combine_body, nv),
        in_specs=[pl.BlockSpec(memory_space=pltpu.SMEM),
                  pl.BlockSpec(memory_space=pltpu.SMEM)],
        out_specs=pl.BlockSpec(memory_space=pltpu.SMEM),
        out_shape=jax.ShapeDtypeStruct((1, 1), jnp.float32),
    )(partials, ncl)
    return out[0, 0]


# speedup vs baseline: 2.2514x; 2.2514x over previous
import functools

import jax
import jax.numpy as jnp
from jax import lax
from jax.experimental import pallas as pl
from jax.experimental.pallas import tpu as pltpu
from jax.experimental.pallas import tpu_sc as plsc

L = 16
NC = 2
NS = 16
NW = NC * NS
BIG = 0x3FFFFFFF
CHUNK = 4000


def _tile_body(preds_hbm, lits_hbm, clauses_hbm, out_hbm,
               preds_v, lits_b0, lits_b1, cls_b0, cls_b1, outbuf_v,
               sem_p, sem_l0, sem_l1, sem_c0, sem_c1,
               *, n_vars, per_tile):
    wid = lax.axis_index("s") * NC + lax.axis_index("c")
    base = wid * per_tile
    nchunk = per_tile // CHUNK
    lits_b = (lits_b0, lits_b1)
    cls_b = (cls_b0, cls_b1)
    sem_l = (sem_l0, sem_l1)
    sem_c = (sem_c0, sem_c1)

    def start_chunk(slot, c):
        off = base + c * CHUNK
        pltpu.make_async_copy(lits_hbm.at[pl.ds(off, CHUNK)],
                              lits_b[slot], sem_l[slot]).start()
        pltpu.make_async_copy(clauses_hbm.at[pl.ds(off, CHUNK)],
                              cls_b[slot], sem_c[slot]).start()

    def wait_chunk(slot):
        pltpu.make_async_copy(lits_hbm.at[pl.ds(0, CHUNK)],
                              lits_b[slot], sem_l[slot]).wait()
        pltpu.make_async_copy(clauses_hbm.at[pl.ds(0, CHUNK)],
                              cls_b[slot], sem_c[slot]).wait()

    preds_cp = pltpu.make_async_copy(preds_hbm, preds_v, sem_p)
    preds_cp.start()
    start_chunk(0, 0)
    start_chunk(1, 1)
    preds_cp.wait()

    iota = lax.iota(jnp.int32, L)
    lane0 = iota == 0
    zero16 = jnp.zeros((L,), jnp.int32)
    one16 = jnp.full((L,), 1, jnp.int32)

    shift_idx = jnp.maximum(iota - 1, 0)
    last_idx = jnp.full((L,), L - 1, jnp.int32)

    def compute(slot, state):
        def vec_body(i, st):
            carry_vec, cnt_vec, first_vec = st
            lit = lits_b[slot][pl.ds(i * L, L)]
            cls = cls_b[slot][pl.ds(i * L, L)]
            is_pos = lit < n_vars
            var = jnp.where(is_pos, lit, lit - n_vars)
            p = plsc.load_gather(preds_v, [var])
            sat = (p >= 0.5) == is_pos
            m = jnp.where(sat, cls, -1)
            incl = plsc.cummax(m)
            shifted = jnp.take_along_axis(incl, shift_idx, axis=0,
                                          mode="promise_in_bounds")
            shifted = jnp.where(iota == 0, -1, shifted)
            excl = jnp.maximum(shifted, carry_vec)
            newc = sat & (cls > excl)
            cnt_vec = cnt_vec + newc.astype(jnp.int32)
            first_vec = jnp.minimum(first_vec, jnp.where(sat, cls, BIG))
            vmax = jnp.take_along_axis(incl, last_idx, axis=0,
                                       mode="promise_in_bounds")
            carry_vec = jnp.maximum(carry_vec, vmax)
            return carry_vec, cnt_vec, first_vec

        return lax.fori_loop(0, CHUNK // L, vec_body, state, unroll=8)

    def one(c, slot, state):
        wait_chunk(slot)
        state = compute(slot, state)

        @pl.when(c + 2 < nchunk)
        def _():
            start_chunk(slot, c + 2)

        return state

    def pair_body(i, state):
        state = one(2 * i, 0, state)
        state = one(2 * i + 1, 1, state)
        return state

    init = (jnp.full((L,), -1, jnp.int32),
            jnp.zeros((L,), jnp.int32),
            jnp.full((L,), BIG, jnp.int32))
    state = lax.fori_loop(0, nchunk // 2, pair_body, init)
    if nchunk % 2:
        state = one(nchunk - 1, 0, state)
    carry_vec, cnt_vec, first_vec = state

    cnt = jnp.sum(cnt_vec)
    first = jnp.min(first_vec)
    last = jnp.max(carry_vec)
    out = jnp.where(iota == 0, cnt,
                    jnp.where(iota == 1, first,
                              jnp.where(iota == 2, last, 0)))
    outbuf_v[...] = out
    pltpu.sync_copy(outbuf_v, out_hbm.at[wid])


def _combine_body(n_vars, partials_ref, ncl_ref, o_ref):
    def body(t, st):
        total, m = st
        c = partials_ref[t, 0]
        f = partials_ref[t, 1]
        l = partials_ref[t, 2]
        dup = jnp.where((c > 0) & (f == m), jnp.int32(1), jnp.int32(0))
        return total + c - dup, jnp.maximum(m, l)

    total, _ = lax.fori_loop(0, NW, body, (jnp.int32(0), jnp.int32(-1)))
    o_ref[0, 0] = (ncl_ref[0, 0] - total.astype(jnp.float32)) / jnp.float32(n_vars)


def kernel(preds, lits, clauses, n_vars, n_clauses):
    del n_vars
    nv = preds.shape[0]
    nnz = lits.shape[0]
    per_tile = nnz // NW
    assert nnz % NW == 0
    assert per_tile % CHUNK == 0 and CHUNK % L == 0

    mesh = plsc.VectorSubcoreMesh(core_axis_name="c", subcore_axis_name="s")
    sc = functools.partial(
        pl.kernel,
        mesh=mesh,
        compiler_params=pltpu.CompilerParams(needs_layout_passes=False),
        out_type=jax.ShapeDtypeStruct((NW, L), jnp.int32),
        scratch_types=[
            pltpu.VMEM((nv,), jnp.float32),
            pltpu.VMEM((CHUNK,), jnp.int32),
            pltpu.VMEM((CHUNK,), jnp.int32),
            pltpu.VMEM((CHUNK,), jnp.int32),
            pltpu.VMEM((CHUNK,), jnp.int32),
            pltpu.VMEM((L,), jnp.int32),
            pltpu.SemaphoreType.DMA,
            pltpu.SemaphoreType.DMA,
            pltpu.SemaphoreType.DMA,
            pltpu.SemaphoreType.DMA,
            pltpu.SemaphoreType.DMA,
        ],
    )(functools.partial(_tile_body, n_vars=nv, per_tile=per_tile))
    partials = sc(preds, lits, clauses)

    ncl = jnp.asarray(n_clauses, jnp.float32).reshape(1, 1)
    out = pl.pallas_call(
        functools.partial(_combine_body, nv),
        in_specs=[pl.BlockSpec(memory_space=pltpu.SMEM),
                  pl.BlockSpec(memory_space=pltpu.SMEM)],
        out_specs=pl.BlockSpec(memory_space=pltpu.SMEM),
        out_shape=jax.ShapeDtypeStruct((1, 1), jnp.float32),
    )(partials, ncl)
    return out[0, 0]
